# baseline (speedup 1.0000x reference)
import functools

import jax
import jax.numpy as jnp
from jax import lax
from jax.experimental import pallas as pl
from jax.experimental.pallas import tpu as pltpu
from jax.experimental.pallas import tpu_sc as plsc

_L = 1_000_000
_NR = 32
_G = 8
_BA = 8_000
_RA = _BA // _G
_NBA = _L // _BA
_GC = 32
_RC = 1_000
_NBC = (_L // _GC + _RC - 1) // _RC
_PREC = lax.Precision.HIGHEST

_BIG_I = 2**30
_INF = float("inf")
_EPS = 2e-6


def _argmin_body(v_ref, w_ref, kb1_ref, kb2_ref, parts_ref):
    step = pl.program_id(0)

    @pl.when(step == 0)
    def _():
        parts_ref[...] = jnp.full((128, 256), _INF, jnp.float32)

    v = v_ref[...]
    c = jnp.concatenate([v, v * v], axis=1)
    s = jnp.dot(c, w_ref[...], precision=_PREC,
                preferred_element_type=jnp.float32)
    parts_ref[pl.ds(step, 1), :] = jnp.min(s, axis=0, keepdims=True)

    @pl.when(step == _NBA - 1)
    def _():
        parts = parts_ref[...]
        bm = parts
        for sh in (32, 64, 128):
            bm = jnp.minimum(bm, pltpu.roll(bm, sh, axis=1))
        gv = jnp.min(bm, axis=0, keepdims=True)
        qual = bm <= gv + _EPS
        rows = lax.broadcasted_iota(jnp.int32, (128, 256), 0)
        kb1 = jnp.min(jnp.where(qual, rows, _BIG_I), axis=0,
                      keepdims=True)
        kb2 = jnp.min(jnp.where(jnp.logical_and(qual, rows > kb1),
                                rows, _BIG_I), axis=0, keepdims=True)
        kb2 = jnp.where(kb2 == _BIG_I, kb1, kb2)
        kb1_ref[...] = jnp.broadcast_to(kb1[0:1, 0:_NR], (8, _NR))
        kb2_ref[...] = jnp.broadcast_to(kb2[0:1, 0:_NR], (8, _NR))


def _make_argmin_call(interpret=False):
  return pl.pallas_call(
    _argmin_body,
    interpret=interpret,
    grid=(_NBA,),
    in_specs=[
        pl.BlockSpec((_RA, 3 * _G), lambda i: (i, 0)),
        pl.BlockSpec((6 * _G, 256), lambda i: (0, 0)),
    ],
    out_specs=[
        pl.BlockSpec((8, _NR), lambda i: (0, 0)),
        pl.BlockSpec((8, _NR), lambda i: (0, 0)),
    ],
    out_shape=[
        jax.ShapeDtypeStruct((8, _NR), jnp.int32),
        jax.ShapeDtypeStruct((8, _NR), jnp.int32),
    ],
    scratch_shapes=[pltpu.VMEM((128, 256), jnp.float32)],
  )


def _copy_body(v_ref, p_ref, big_ref):
    big_ref[...] = jnp.dot(v_ref[...], p_ref[...], precision=_PREC,
                           preferred_element_type=jnp.float32)


def _make_copy_call(interpret=False):
  return pl.pallas_call(
    _copy_body,
    interpret=interpret,
    grid=(_NBC,),
    in_specs=[
        pl.BlockSpec((_RC, 3 * _GC), lambda i: (i, 0)),
        pl.BlockSpec((3 * _GC, 4 * _GC), lambda i: (0, 0)),
    ],
    out_specs=pl.BlockSpec((_RC, 4 * _GC), lambda i: (i, 0)),
    out_shape=jax.ShapeDtypeStruct((_L // _GC, 4 * _GC), jnp.float32),
  )


def _extract_body(kbc_ref, v_ref, rt_ref, idx_ref, cx_ref, cy_ref,
                  cz_ref, bd_ref, ai_ref, ax_ref, ay_ref, az_ref):
    q = pl.program_id(0)
    b = pl.program_id(1)

    v = v_ref[...]
    row24 = lax.broadcasted_iota(jnp.int32, (_RA, 3 * _G), 0)
    lane24 = lax.broadcasted_iota(jnp.int32, (_RA, 3 * _G), 1)
    rrow_iota = lax.broadcasted_iota(jnp.int32, (_NR, 3 * _G), 0)
    rrow = jnp.sum(jnp.where(rrow_iota == b, rt_ref[...], 0.0),
                   axis=0, keepdims=True)
    dd = v - rrow
    d2 = dd * dd
    s3 = d2 + pltpu.roll(d2, 3 * _G - 1, axis=1) + pltpu.roll(d2, 3 * _G - 2, axis=1)
    crd = lane24 - (lane24 // 3) * 3
    d2p = jnp.where(crd == 0, s3, _INF)
    m = jnp.min(d2p)
    flatid = row24 * _G + lane24 // 3
    flat = jnp.min(jnp.where(d2p == m, flatid, _BIG_I))
    kblk = kbc_ref[q * _NR + b]
    gidx = kblk * _BA + flat
    rstar = flat // _G
    gstar = flat - rstar * _G
    wsel = jnp.logical_and(row24 == rstar, lane24 // 3 == gstar)
    zero = jnp.zeros((), jnp.float32)
    sx = jnp.sum(jnp.where(jnp.logical_and(wsel, crd == 0), v, zero))
    sy = jnp.sum(jnp.where(jnp.logical_and(wsel, crd == 1), v, zero))
    sz = jnp.sum(jnp.where(jnp.logical_and(wsel, crd == 2), v, zero))
    lane32 = lax.broadcasted_iota(jnp.int32, (1, _NR), 1)
    isb = lane32 == b

    @pl.when(q == 0)
    def _():
        bd_ref[...] = jnp.where(isb, m, bd_ref[...])
        ai_ref[...] = jnp.where(isb, gidx, ai_ref[...])
        ax_ref[...] = jnp.where(isb, sx, ax_ref[...])
        ay_ref[...] = jnp.where(isb, sy, ay_ref[...])
        az_ref[...] = jnp.where(isb, sz, az_ref[...])

    @pl.when(q == 1)
    def _():
        better = jnp.logical_or(
            m < bd_ref[...],
            jnp.logical_and(m == bd_ref[...], gidx < ai_ref[...]))
        upd = jnp.logical_and(isb, better)
        bd_ref[...] = jnp.where(upd, m, bd_ref[...])
        ai_ref[...] = jnp.where(upd, gidx, ai_ref[...])
        ax_ref[...] = jnp.where(upd, sx, ax_ref[...])
        ay_ref[...] = jnp.where(upd, sy, ay_ref[...])
        az_ref[...] = jnp.where(upd, sz, az_ref[...])

    @pl.when(jnp.logical_and(q == 1, b == _NR - 1))
    def _():
        idx_ref[...] = jnp.broadcast_to(ai_ref[...], (8, _NR))
        cx_ref[...] = jnp.broadcast_to(ax_ref[...], (8, _NR))
        cy_ref[...] = jnp.broadcast_to(ay_ref[...], (8, _NR))
        cz_ref[...] = jnp.broadcast_to(az_ref[...], (8, _NR))


def _make_extract_call(interpret=False):
  return pl.pallas_call(
    _extract_body,
    interpret=interpret,
    grid_spec=pltpu.PrefetchScalarGridSpec(
        num_scalar_prefetch=1,
        grid=(2, _NR),
        in_specs=[
            pl.BlockSpec((_RA, 3 * _G),
                         lambda q, b, kbc: (kbc[q * _NR + b], 0)),
            pl.BlockSpec((_NR, 3 * _G), lambda q, b, kbc: (0, 0)),
        ],
        out_specs=[
            pl.BlockSpec((8, _NR), lambda q, b, kbc: (0, 0)),
            pl.BlockSpec((8, _NR), lambda q, b, kbc: (0, 0)),
            pl.BlockSpec((8, _NR), lambda q, b, kbc: (0, 0)),
            pl.BlockSpec((8, _NR), lambda q, b, kbc: (0, 0)),
        ],
        scratch_shapes=[
            pltpu.VMEM((1, _NR), jnp.float32),
            pltpu.VMEM((1, _NR), jnp.int32),
            pltpu.VMEM((1, _NR), jnp.float32),
            pltpu.VMEM((1, _NR), jnp.float32),
            pltpu.VMEM((1, _NR), jnp.float32),
        ],
    ),
    out_shape=[
        jax.ShapeDtypeStruct((8, _NR), jnp.int32),
        jax.ShapeDtypeStruct((8, _NR), jnp.float32),
        jax.ShapeDtypeStruct((8, _NR), jnp.float32),
        jax.ShapeDtypeStruct((8, _NR), jnp.float32),
    ],
  )


def _sc_scatter_body(idx_hbm, rows_hbm, big_ref, idx_v, rows_v, sem):
    wid = lax.axis_index("c") * 16 + lax.axis_index("s")

    @pl.when(wid == 0)
    def _():
        pltpu.sync_copy(idx_hbm, idx_v)
        pltpu.sync_copy(rows_hbm, rows_v)
        vecs = [idx_v[pl.ds(0, 16)], idx_v[pl.ds(16, 16)]]
        copies = []
        for j in range(_NR):
            rowid = vecs[j // 16][j % 16]
            copies.append(pltpu.async_copy(
                rows_v.at[pl.ds(j, 1), :],
                big_ref.at[pl.ds(rowid, 1), :],
                sem))
        for cc in copies:
            cc.wait()


@functools.lru_cache(maxsize=None)
def _make_sc_scatter():
    mesh = plsc.VectorSubcoreMesh(core_axis_name="c", subcore_axis_name="s")
    return pl.kernel(
        _sc_scatter_body,
        out_type=(),
        mesh=mesh,
        scratch_types=[
            pltpu.VMEM((_NR,), jnp.int32),
            pltpu.VMEM((_NR, 4), jnp.float32),
            pltpu.SemaphoreType.DMA,
        ],
    )


def _build_w(receiver_pos):
    r = receiver_pos.astype(jnp.float32)
    eye_g = jnp.eye(_G, dtype=jnp.float32)
    w1 = (eye_g[:, None, :, None]
          * (-2.0 * r.T)[None, :, None, :])
    w2 = (eye_g[:, None, :, None]
          * jnp.ones((1, 3, 1, _NR), jnp.float32))
    return jnp.concatenate(
        [w1.reshape(3 * _G, 256), w2.reshape(3 * _G, 256)], axis=0)


def _build_p():
    eye_q = jnp.eye(_GC, dtype=jnp.float32)
    e34 = jnp.eye(3, 4, dtype=jnp.float32)
    return (eye_q[:, None, :, None] * e34[None, :, None, :]).reshape(
        3 * _GC, 4 * _GC)


def kernel(mesh_3D, receiver_pos):
    mesh8 = mesh_3D.reshape(_L // _G, 3 * _G)
    mesh32 = mesh_3D.reshape(_L // _GC, 3 * _GC)
    w = _build_w(receiver_pos)
    rt = jnp.tile(receiver_pos.astype(jnp.float32), (1, _G))
    kb1_8, kb2_8 = _make_argmin_call()(mesh8, w)
    kbc = jnp.concatenate([kb1_8[0], kb2_8[0]])
    idx8, cx8, cy8, cz8 = _make_extract_call()(kbc, mesh8, rt)
    big32 = _make_copy_call()(mesh32, _build_p())
    big = big32.reshape(_L, 4)
    idx = idx8[0]
    closest = jnp.stack([cx8[0], cy8[0], cz8[0]], axis=1)
    rows = jnp.concatenate(
        [closest, jnp.ones((_NR, 1), jnp.float32)], axis=1)
    big_ref = jax.new_ref(big)
    _make_sc_scatter()(idx, rows, big_ref)
    return big_ref[...], closest


# speedup vs baseline: 1.0567x; 1.0567x over previous
import functools

import jax
import jax.numpy as jnp
from jax import lax
from jax.experimental import pallas as pl
from jax.experimental.pallas import tpu as pltpu
from jax.experimental.pallas import tpu_sc as plsc

_L = 1_000_000
_NR = 32
_G = 8
_BA = 8_000
_RA = _BA // _G
_NBA = _L // _BA
_GC = 32
_RC = 1_000
_NBC = (_L // _GC + _RC - 1) // _RC
_PREC = lax.Precision.HIGHEST

_BIG_I = 2**30
_INF = float("inf")
_EPS = 2e-6


def _argmin_body(v_ref, w_ref, kb1_ref, kb2_ref, parts_ref):
    step = pl.program_id(0)

    @pl.when(step == 0)
    def _():
        parts_ref[...] = jnp.full((128, 256), _INF, jnp.float32)

    v = v_ref[...]
    c = jnp.concatenate([v, v * v], axis=1)
    s = jnp.dot(c, w_ref[...], precision=_PREC,
                preferred_element_type=jnp.float32)
    parts_ref[pl.ds(step, 1), :] = jnp.min(s, axis=0, keepdims=True)

    @pl.when(step == _NBA - 1)
    def _():
        parts = parts_ref[...]
        bm = parts
        for sh in (32, 64, 128):
            bm = jnp.minimum(bm, pltpu.roll(bm, sh, axis=1))
        gv = jnp.min(bm, axis=0, keepdims=True)
        qual = bm <= gv + _EPS
        rows = lax.broadcasted_iota(jnp.int32, (128, 256), 0)
        kb1 = jnp.min(jnp.where(qual, rows, _BIG_I), axis=0,
                      keepdims=True)
        kb2 = jnp.min(jnp.where(jnp.logical_and(qual, rows > kb1),
                                rows, _BIG_I), axis=0, keepdims=True)
        kb2 = jnp.where(kb2 == _BIG_I, kb1, kb2)
        kb1_ref[...] = jnp.broadcast_to(kb1[0:1, 0:_NR], (8, _NR))
        kb2_ref[...] = jnp.broadcast_to(kb2[0:1, 0:_NR], (8, _NR))


def _make_argmin_call(interpret=False):
  return pl.pallas_call(
    _argmin_body,
    interpret=interpret,
    grid=(_NBA,),
    in_specs=[
        pl.BlockSpec((_RA, 3 * _G), lambda i: (i, 0)),
        pl.BlockSpec((6 * _G, 256), lambda i: (0, 0)),
    ],
    out_specs=[
        pl.BlockSpec((8, _NR), lambda i: (0, 0)),
        pl.BlockSpec((8, _NR), lambda i: (0, 0)),
    ],
    out_shape=[
        jax.ShapeDtypeStruct((8, _NR), jnp.int32),
        jax.ShapeDtypeStruct((8, _NR), jnp.int32),
    ],
    scratch_shapes=[pltpu.VMEM((128, 256), jnp.float32)],
  )


_BC = 8_000
_NBC2 = _L // _BC


def _copy_body(blk_ref, big_ref):
    blk = blk_ref[...]
    big_ref[:, 0:3] = blk
    big_ref[:, 3:4] = jnp.zeros((_BC, 1), jnp.float32)


def _make_copy_call(interpret=False):
  return pl.pallas_call(
    _copy_body,
    interpret=interpret,
    grid=(_NBC2,),
    in_specs=[pl.BlockSpec((_BC, 3), lambda i: (i, 0))],
    out_specs=pl.BlockSpec((_BC, 4), lambda i: (i, 0)),
    out_shape=jax.ShapeDtypeStruct((_L, 4), jnp.float32),
  )


def _extract_body(kbc_ref, v_ref, rt_ref, idx_ref, cx_ref, cy_ref,
                  cz_ref, bd_ref, ai_ref, ax_ref, ay_ref, az_ref):
    q = pl.program_id(0)
    b = pl.program_id(1)

    v = v_ref[...]
    row24 = lax.broadcasted_iota(jnp.int32, (_RA, 3 * _G), 0)
    lane24 = lax.broadcasted_iota(jnp.int32, (_RA, 3 * _G), 1)
    rrow_iota = lax.broadcasted_iota(jnp.int32, (_NR, 3 * _G), 0)
    rrow = jnp.sum(jnp.where(rrow_iota == b, rt_ref[...], 0.0),
                   axis=0, keepdims=True)
    dd = v - rrow
    d2 = dd * dd
    s3 = d2 + pltpu.roll(d2, 3 * _G - 1, axis=1) + pltpu.roll(d2, 3 * _G - 2, axis=1)
    crd = lane24 - (lane24 // 3) * 3
    d2p = jnp.where(crd == 0, s3, _INF)
    m = jnp.min(d2p)
    flatid = row24 * _G + lane24 // 3
    flat = jnp.min(jnp.where(d2p == m, flatid, _BIG_I))
    kblk = kbc_ref[q * _NR + b]
    gidx = kblk * _BA + flat
    rstar = flat // _G
    gstar = flat - rstar * _G
    wsel = jnp.logical_and(row24 == rstar, lane24 // 3 == gstar)
    zero = jnp.zeros((), jnp.float32)
    sx = jnp.sum(jnp.where(jnp.logical_and(wsel, crd == 0), v, zero))
    sy = jnp.sum(jnp.where(jnp.logical_and(wsel, crd == 1), v, zero))
    sz = jnp.sum(jnp.where(jnp.logical_and(wsel, crd == 2), v, zero))
    lane32 = lax.broadcasted_iota(jnp.int32, (1, _NR), 1)
    isb = lane32 == b

    @pl.when(q == 0)
    def _():
        bd_ref[...] = jnp.where(isb, m, bd_ref[...])
        ai_ref[...] = jnp.where(isb, gidx, ai_ref[...])
        ax_ref[...] = jnp.where(isb, sx, ax_ref[...])
        ay_ref[...] = jnp.where(isb, sy, ay_ref[...])
        az_ref[...] = jnp.where(isb, sz, az_ref[...])

    @pl.when(q == 1)
    def _():
        better = jnp.logical_or(
            m < bd_ref[...],
            jnp.logical_and(m == bd_ref[...], gidx < ai_ref[...]))
        upd = jnp.logical_and(isb, better)
        bd_ref[...] = jnp.where(upd, m, bd_ref[...])
        ai_ref[...] = jnp.where(upd, gidx, ai_ref[...])
        ax_ref[...] = jnp.where(upd, sx, ax_ref[...])
        ay_ref[...] = jnp.where(upd, sy, ay_ref[...])
        az_ref[...] = jnp.where(upd, sz, az_ref[...])

    @pl.when(jnp.logical_and(q == 1, b == _NR - 1))
    def _():
        idx_ref[...] = jnp.broadcast_to(ai_ref[...], (8, _NR))
        cx_ref[...] = jnp.broadcast_to(ax_ref[...], (8, _NR))
        cy_ref[...] = jnp.broadcast_to(ay_ref[...], (8, _NR))
        cz_ref[...] = jnp.broadcast_to(az_ref[...], (8, _NR))


def _make_extract_call(interpret=False):
  return pl.pallas_call(
    _extract_body,
    interpret=interpret,
    grid_spec=pltpu.PrefetchScalarGridSpec(
        num_scalar_prefetch=1,
        grid=(2, _NR),
        in_specs=[
            pl.BlockSpec((_RA, 3 * _G),
                         lambda q, b, kbc: (kbc[q * _NR + b], 0)),
            pl.BlockSpec((_NR, 3 * _G), lambda q, b, kbc: (0, 0)),
        ],
        out_specs=[
            pl.BlockSpec((8, _NR), lambda q, b, kbc: (0, 0)),
            pl.BlockSpec((8, _NR), lambda q, b, kbc: (0, 0)),
            pl.BlockSpec((8, _NR), lambda q, b, kbc: (0, 0)),
            pl.BlockSpec((8, _NR), lambda q, b, kbc: (0, 0)),
        ],
        scratch_shapes=[
            pltpu.VMEM((1, _NR), jnp.float32),
            pltpu.VMEM((1, _NR), jnp.int32),
            pltpu.VMEM((1, _NR), jnp.float32),
            pltpu.VMEM((1, _NR), jnp.float32),
            pltpu.VMEM((1, _NR), jnp.float32),
        ],
    ),
    out_shape=[
        jax.ShapeDtypeStruct((8, _NR), jnp.int32),
        jax.ShapeDtypeStruct((8, _NR), jnp.float32),
        jax.ShapeDtypeStruct((8, _NR), jnp.float32),
        jax.ShapeDtypeStruct((8, _NR), jnp.float32),
    ],
  )


def _sc_scatter_body(idx_hbm, rows_hbm, big_ref, idx_v, rows_v, sem):
    wid = lax.axis_index("c") * 16 + lax.axis_index("s")

    @pl.when(wid == 0)
    def _():
        pltpu.sync_copy(idx_hbm, idx_v)
        pltpu.sync_copy(rows_hbm, rows_v)
        vecs = [idx_v[pl.ds(0, 16)], idx_v[pl.ds(16, 16)]]
        copies = []
        for j in range(_NR):
            rowid = vecs[j // 16][j % 16]
            copies.append(pltpu.async_copy(
                rows_v.at[pl.ds(j, 1), :],
                big_ref.at[pl.ds(rowid, 1), :],
                sem))
        for cc in copies:
            cc.wait()


@functools.lru_cache(maxsize=None)
def _make_sc_scatter():
    mesh = plsc.VectorSubcoreMesh(core_axis_name="c", subcore_axis_name="s")
    return pl.kernel(
        _sc_scatter_body,
        out_type=(),
        mesh=mesh,
        scratch_types=[
            pltpu.VMEM((_NR,), jnp.int32),
            pltpu.VMEM((_NR, 4), jnp.float32),
            pltpu.SemaphoreType.DMA,
        ],
    )


def _build_w(receiver_pos):
    r = receiver_pos.astype(jnp.float32)
    eye_g = jnp.eye(_G, dtype=jnp.float32)
    w1 = (eye_g[:, None, :, None]
          * (-2.0 * r.T)[None, :, None, :])
    w2 = (eye_g[:, None, :, None]
          * jnp.ones((1, 3, 1, _NR), jnp.float32))
    return jnp.concatenate(
        [w1.reshape(3 * _G, 256), w2.reshape(3 * _G, 256)], axis=0)


def _build_p():
    eye_q = jnp.eye(_GC, dtype=jnp.float32)
    e34 = jnp.eye(3, 4, dtype=jnp.float32)
    return (eye_q[:, None, :, None] * e34[None, :, None, :]).reshape(
        3 * _GC, 4 * _GC)


def kernel(mesh_3D, receiver_pos):
    mesh8 = mesh_3D.reshape(_L // _G, 3 * _G)
    w = _build_w(receiver_pos)
    rt = jnp.tile(receiver_pos.astype(jnp.float32), (1, _G))
    kb1_8, kb2_8 = _make_argmin_call()(mesh8, w)
    kbc = jnp.concatenate([kb1_8[0], kb2_8[0]])
    idx8, cx8, cy8, cz8 = _make_extract_call()(kbc, mesh8, rt)
    big = _make_copy_call()(mesh_3D.reshape(_L, 3))
    idx = idx8[0]
    closest = jnp.stack([cx8[0], cy8[0], cz8[0]], axis=1)
    rows = jnp.concatenate(
        [closest, jnp.ones((_NR, 1), jnp.float32)], axis=1)
    big_ref = jax.new_ref(big)
    _make_sc_scatter()(idx, rows, big_ref)
    return big_ref[...], closest


# speedup vs baseline: 1.0648x; 1.0077x over previous
import functools

import jax
import jax.numpy as jnp
from jax import lax
from jax.experimental import pallas as pl
from jax.experimental.pallas import tpu as pltpu
from jax.experimental.pallas import tpu_sc as plsc

_L = 1_000_000
_NR = 32
_G = 8
_RA = 1_024
_NROWS = _L // _G
_NBA = -(-_NROWS // _RA)
_CR = 256
_CB = _CR * _G
_NCH = -(-_NROWS // _CR)
_GC = 32
_RC = 1_000
_NBC = (_L // _GC + _RC - 1) // _RC
_PREC = lax.Precision.HIGHEST

_BIG_I = 2**30
_INF = float("inf")
_EPS = 2e-6


def _argmin_body(v_ref, w_ref, kb1_ref, kb2_ref, parts_ref):
    step = pl.program_id(0)

    @pl.when(step == 0)
    def _():
        parts_ref[...] = jnp.full((512, 256), _INF, jnp.float32)

    v = v_ref[...]
    c = jnp.concatenate([v, v * v], axis=1)
    s = jnp.dot(c, w_ref[...], precision=_PREC,
                preferred_element_type=jnp.float32)
    grow = lax.broadcasted_iota(jnp.int32, (_RA, 256), 0) + step * _RA
    s = jnp.where(grow < _NROWS, s, _INF)
    for j in range(_RA // _CR):
        parts_ref[pl.ds(step * (_RA // _CR) + j, 1), :] = jnp.min(
            s[j * _CR:(j + 1) * _CR], axis=0, keepdims=True)

    @pl.when(step == _NBA - 1)
    def _():
        parts = parts_ref[...]
        bm = parts
        for sh in (32, 64, 128):
            bm = jnp.minimum(bm, pltpu.roll(bm, sh, axis=1))
        gv = jnp.min(bm, axis=0, keepdims=True)
        qual = bm <= gv + _EPS
        rows = lax.broadcasted_iota(jnp.int32, (512, 256), 0)
        kb1 = jnp.min(jnp.where(qual, rows, _BIG_I), axis=0,
                      keepdims=True)
        kb2 = jnp.min(jnp.where(jnp.logical_and(qual, rows > kb1),
                                rows, _BIG_I), axis=0, keepdims=True)
        kb2 = jnp.where(kb2 == _BIG_I, kb1, kb2)
        kb1_ref[...] = jnp.broadcast_to(kb1[0:1, 0:_NR], (8, _NR))
        kb2_ref[...] = jnp.broadcast_to(kb2[0:1, 0:_NR], (8, _NR))


def _make_argmin_call(interpret=False):
  return pl.pallas_call(
    _argmin_body,
    interpret=interpret,
    grid=(_NBA,),
    in_specs=[
        pl.BlockSpec((_RA, 3 * _G), lambda i: (i, 0)),
        pl.BlockSpec((6 * _G, 256), lambda i: (0, 0)),
    ],
    out_specs=[
        pl.BlockSpec((8, _NR), lambda i: (0, 0)),
        pl.BlockSpec((8, _NR), lambda i: (0, 0)),
    ],
    out_shape=[
        jax.ShapeDtypeStruct((8, _NR), jnp.int32),
        jax.ShapeDtypeStruct((8, _NR), jnp.int32),
    ],
    scratch_shapes=[pltpu.VMEM((512, 256), jnp.float32)],
  )


_BC = 8_000
_NBC2 = _L // _BC


def _copy_body(blk_ref, big_ref):
    blk = blk_ref[...]
    big_ref[:, 0:3] = blk
    big_ref[:, 3:4] = jnp.zeros((_BC, 1), jnp.float32)


def _make_copy_call(interpret=False):
  return pl.pallas_call(
    _copy_body,
    interpret=interpret,
    grid=(_NBC2,),
    in_specs=[pl.BlockSpec((_BC, 3), lambda i: (i, 0))],
    out_specs=pl.BlockSpec((_BC, 4), lambda i: (i, 0)),
    out_shape=jax.ShapeDtypeStruct((_L, 4), jnp.float32),
  )


def _extract_body(kbc_ref, v_ref, rt_ref, idx_ref, cx_ref, cy_ref,
                  cz_ref, bd_ref, ai_ref, ax_ref, ay_ref, az_ref):
    q = pl.program_id(0)
    b = pl.program_id(1)

    v = v_ref[...]
    row24 = lax.broadcasted_iota(jnp.int32, (_CR, 3 * _G), 0)
    lane24 = lax.broadcasted_iota(jnp.int32, (_CR, 3 * _G), 1)
    rrow_iota = lax.broadcasted_iota(jnp.int32, (_NR, 3 * _G), 0)
    rrow = jnp.sum(jnp.where(rrow_iota == b, rt_ref[...], 0.0),
                   axis=0, keepdims=True)
    kblk = kbc_ref[q * _NR + b]
    dd = v - rrow
    d2 = dd * dd
    s3 = d2 + pltpu.roll(d2, 3 * _G - 1, axis=1) + pltpu.roll(d2, 3 * _G - 2, axis=1)
    crd = lane24 - (lane24 // 3) * 3
    valid = jnp.logical_and(crd == 0, row24 + kblk * _CR < _NROWS)
    d2p = jnp.where(valid, s3, _INF)
    m = jnp.min(d2p)
    flatid = row24 * _G + lane24 // 3
    flat = jnp.min(jnp.where(d2p == m, flatid, _BIG_I))
    gidx = kblk * _CB + flat
    rstar = flat // _G
    gstar = flat - rstar * _G
    wsel = jnp.logical_and(row24 == rstar, lane24 // 3 == gstar)
    zero = jnp.zeros((), jnp.float32)
    sx = jnp.sum(jnp.where(jnp.logical_and(wsel, crd == 0), v, zero))
    sy = jnp.sum(jnp.where(jnp.logical_and(wsel, crd == 1), v, zero))
    sz = jnp.sum(jnp.where(jnp.logical_and(wsel, crd == 2), v, zero))
    lane32 = lax.broadcasted_iota(jnp.int32, (1, _NR), 1)
    isb = lane32 == b

    @pl.when(q == 0)
    def _():
        bd_ref[...] = jnp.where(isb, m, bd_ref[...])
        ai_ref[...] = jnp.where(isb, gidx, ai_ref[...])
        ax_ref[...] = jnp.where(isb, sx, ax_ref[...])
        ay_ref[...] = jnp.where(isb, sy, ay_ref[...])
        az_ref[...] = jnp.where(isb, sz, az_ref[...])

    @pl.when(q == 1)
    def _():
        better = jnp.logical_or(
            m < bd_ref[...],
            jnp.logical_and(m == bd_ref[...], gidx < ai_ref[...]))
        upd = jnp.logical_and(isb, better)
        bd_ref[...] = jnp.where(upd, m, bd_ref[...])
        ai_ref[...] = jnp.where(upd, gidx, ai_ref[...])
        ax_ref[...] = jnp.where(upd, sx, ax_ref[...])
        ay_ref[...] = jnp.where(upd, sy, ay_ref[...])
        az_ref[...] = jnp.where(upd, sz, az_ref[...])

    @pl.when(jnp.logical_and(q == 1, b == _NR - 1))
    def _():
        idx_ref[...] = jnp.broadcast_to(ai_ref[...], (8, _NR))
        cx_ref[...] = jnp.broadcast_to(ax_ref[...], (8, _NR))
        cy_ref[...] = jnp.broadcast_to(ay_ref[...], (8, _NR))
        cz_ref[...] = jnp.broadcast_to(az_ref[...], (8, _NR))


def _make_extract_call(interpret=False):
  return pl.pallas_call(
    _extract_body,
    interpret=interpret,
    grid_spec=pltpu.PrefetchScalarGridSpec(
        num_scalar_prefetch=1,
        grid=(2, _NR),
        in_specs=[
            pl.BlockSpec((_CR, 3 * _G),
                         lambda q, b, kbc: (kbc[q * _NR + b], 0)),
            pl.BlockSpec((_NR, 3 * _G), lambda q, b, kbc: (0, 0)),
        ],
        out_specs=[
            pl.BlockSpec((8, _NR), lambda q, b, kbc: (0, 0)),
            pl.BlockSpec((8, _NR), lambda q, b, kbc: (0, 0)),
            pl.BlockSpec((8, _NR), lambda q, b, kbc: (0, 0)),
            pl.BlockSpec((8, _NR), lambda q, b, kbc: (0, 0)),
        ],
        scratch_shapes=[
            pltpu.VMEM((1, _NR), jnp.float32),
            pltpu.VMEM((1, _NR), jnp.int32),
            pltpu.VMEM((1, _NR), jnp.float32),
            pltpu.VMEM((1, _NR), jnp.float32),
            pltpu.VMEM((1, _NR), jnp.float32),
        ],
    ),
    out_shape=[
        jax.ShapeDtypeStruct((8, _NR), jnp.int32),
        jax.ShapeDtypeStruct((8, _NR), jnp.float32),
        jax.ShapeDtypeStruct((8, _NR), jnp.float32),
        jax.ShapeDtypeStruct((8, _NR), jnp.float32),
    ],
  )


def _sc_scatter_body(idx_hbm, rows_hbm, big_ref, idx_v, rows_v, sem):
    wid = lax.axis_index("c") * 16 + lax.axis_index("s")

    @pl.when(wid == 0)
    def _():
        pltpu.sync_copy(idx_hbm, idx_v)
        pltpu.sync_copy(rows_hbm, rows_v)
        vecs = [idx_v[pl.ds(0, 16)], idx_v[pl.ds(16, 16)]]
        copies = []
        for j in range(_NR):
            rowid = vecs[j // 16][j % 16]
            copies.append(pltpu.async_copy(
                rows_v.at[pl.ds(j, 1), :],
                big_ref.at[pl.ds(rowid, 1), :],
                sem))
        for cc in copies:
            cc.wait()


@functools.lru_cache(maxsize=None)
def _make_sc_scatter():
    mesh = plsc.VectorSubcoreMesh(core_axis_name="c", subcore_axis_name="s")
    return pl.kernel(
        _sc_scatter_body,
        out_type=(),
        mesh=mesh,
        scratch_types=[
            pltpu.VMEM((_NR,), jnp.int32),
            pltpu.VMEM((_NR, 4), jnp.float32),
            pltpu.SemaphoreType.DMA,
        ],
    )


def _build_w(receiver_pos):
    r = receiver_pos.astype(jnp.float32)
    eye_g = jnp.eye(_G, dtype=jnp.float32)
    w1 = (eye_g[:, None, :, None]
          * (-2.0 * r.T)[None, :, None, :])
    w2 = (eye_g[:, None, :, None]
          * jnp.ones((1, 3, 1, _NR), jnp.float32))
    return jnp.concatenate(
        [w1.reshape(3 * _G, 256), w2.reshape(3 * _G, 256)], axis=0)


def _build_p():
    eye_q = jnp.eye(_GC, dtype=jnp.float32)
    e34 = jnp.eye(3, 4, dtype=jnp.float32)
    return (eye_q[:, None, :, None] * e34[None, :, None, :]).reshape(
        3 * _GC, 4 * _GC)


def kernel(mesh_3D, receiver_pos):
    one = jnp.float32(1.0) + jnp.float32(0.0) * receiver_pos[0, 0]
    mesh8 = mesh_3D.reshape(_L // _G, 3 * _G) * one
    w = _build_w(receiver_pos)
    rt = jnp.tile(receiver_pos.astype(jnp.float32), (1, _G))
    kb1_8, kb2_8 = _make_argmin_call()(mesh8, w)
    kbc = jnp.concatenate([kb1_8[0], kb2_8[0]])
    idx8, cx8, cy8, cz8 = _make_extract_call()(kbc, mesh8, rt)
    big = _make_copy_call()(mesh_3D.reshape(_L, 3))
    idx = idx8[0]
    closest = jnp.stack([cx8[0], cy8[0], cz8[0]], axis=1)
    rows = jnp.concatenate(
        [closest, jnp.ones((_NR, 1), jnp.float32)], axis=1)
    big_ref = jax.new_ref(big)
    _make_sc_scatter()(idx, rows, big_ref)
    return big_ref[...], closest


# speedup vs baseline: 14.7846x; 13.8848x over previous
import functools

import jax
import jax.numpy as jnp
from jax import lax
from jax.experimental import pallas as pl
from jax.experimental.pallas import tpu as pltpu
from jax.experimental.pallas import tpu_sc as plsc

_L = 1_000_000
_NR = 32
_BL = 16_384
_NBA = -(-_L // _BL)
_CL = 2_048
_CPB = _BL // _CL
_PREC = lax.Precision.HIGHEST

_BIG_I = 2**30
_INF = float("inf")
_FAR = 3.0e9


def _argmin_body(mt_ref, w_ref, kb1_ref, kb2_ref,
                 b1v_ref, b1c_ref, b2v_ref, b2c_ref):
    step = pl.program_id(0)

    @pl.when(step == 0)
    def _():
        b1v_ref[...] = jnp.full((_NR, 8), _INF, jnp.float32)
        b1c_ref[...] = jnp.zeros((_NR, 8), jnp.int32)
        b2v_ref[...] = jnp.full((_NR, 8), _INF, jnp.float32)
        b2c_ref[...] = jnp.zeros((_NR, 8), jnp.int32)

    lane = lax.broadcasted_iota(jnp.int32, (3, _BL), 1)
    ok = lane + step * _BL < _L
    blk = jnp.where(ok, mt_ref[...], _FAR)
    p6 = jnp.concatenate([blk, blk * blk], axis=0)
    s = jnp.dot(w_ref[:, 0:6], p6, precision=_PREC,
                preferred_element_type=jnp.float32)
    for j in range(_CPB):
        m = jnp.min(s[:, j * _CL:(j + 1) * _CL], axis=1,
                    keepdims=True)
        m = jnp.broadcast_to(m, (_NR, 8))
        cid = step * _CPB + j
        is1 = m < b1v_ref[...]
        is2 = m < b2v_ref[...]
        b2v_ref[...] = jnp.where(is1, b1v_ref[...],
                                 jnp.where(is2, m, b2v_ref[...]))
        b2c_ref[...] = jnp.where(is1, b1c_ref[...],
                                 jnp.where(is2, cid, b2c_ref[...]))
        b1v_ref[...] = jnp.where(is1, m, b1v_ref[...])
        b1c_ref[...] = jnp.where(is1, cid, b1c_ref[...])

    @pl.when(step == _NBA - 1)
    def _():
        kb1_ref[...] = b1c_ref[...]
        kb2_ref[...] = b2c_ref[...]


_argmin_call = pl.pallas_call(
    _argmin_body,
    grid=(_NBA,),
    in_specs=[
        pl.BlockSpec((3, _BL), lambda i: (0, i)),
        pl.BlockSpec((_NR, 8), lambda i: (0, 0)),
    ],
    out_specs=[
        pl.BlockSpec((_NR, 8), lambda i: (0, 0)),
        pl.BlockSpec((_NR, 8), lambda i: (0, 0)),
    ],
    out_shape=[
        jax.ShapeDtypeStruct((_NR, 8), jnp.int32),
        jax.ShapeDtypeStruct((_NR, 8), jnp.int32),
    ],
    scratch_shapes=[
        pltpu.VMEM((_NR, 8), jnp.float32),
        pltpu.VMEM((_NR, 8), jnp.int32),
        pltpu.VMEM((_NR, 8), jnp.float32),
        pltpu.VMEM((_NR, 8), jnp.int32),
    ],
)


def _extract_body(kbc_ref, mt_ref, rt_ref, idx_ref, cx_ref, cy_ref,
                  cz_ref, bd_ref, ai_ref, ax_ref, ay_ref, az_ref):
    q = pl.program_id(0)
    b = pl.program_id(1)

    kblk = kbc_ref[q * _NR + b]
    lane = lax.broadcasted_iota(jnp.int32, (3, _CL), 1)
    blk = mt_ref[...]
    lane32 = lax.broadcasted_iota(jnp.int32, (8, _NR), 1)
    rcol = jnp.sum(jnp.where(lane32 == b, rt_ref[...], 0.0),
                   axis=1, keepdims=True)[0:3]
    dd = blk - rcol
    d2 = dd * dd
    d2p = (d2[0:1] + d2[1:2]) + d2[2:3]
    d2p = jnp.where(lane[0:1] + kblk * _CL < _L, d2p, _INF)
    m = jnp.min(d2p)
    flat = jnp.min(jnp.where(d2p == m, lane[0:1], _BIG_I))
    gidx = kblk * _CL + flat
    wsel = lane[0:1] == flat
    zero = jnp.zeros((), jnp.float32)
    sx = jnp.sum(jnp.where(wsel, blk[0:1], zero))
    sy = jnp.sum(jnp.where(wsel, blk[1:2], zero))
    sz = jnp.sum(jnp.where(wsel, blk[2:3], zero))
    lane1 = lax.broadcasted_iota(jnp.int32, (1, _NR), 1)
    isb = lane1 == b

    @pl.when(q == 0)
    def _():
        bd_ref[...] = jnp.where(isb, m, bd_ref[...])
        ai_ref[...] = jnp.where(isb, gidx, ai_ref[...])
        ax_ref[...] = jnp.where(isb, sx, ax_ref[...])
        ay_ref[...] = jnp.where(isb, sy, ay_ref[...])
        az_ref[...] = jnp.where(isb, sz, az_ref[...])

    @pl.when(q == 1)
    def _():
        better = jnp.logical_or(
            m < bd_ref[...],
            jnp.logical_and(m == bd_ref[...], gidx < ai_ref[...]))
        upd = jnp.logical_and(isb, better)
        bd_ref[...] = jnp.where(upd, m, bd_ref[...])
        ai_ref[...] = jnp.where(upd, gidx, ai_ref[...])
        ax_ref[...] = jnp.where(upd, sx, ax_ref[...])
        ay_ref[...] = jnp.where(upd, sy, ay_ref[...])
        az_ref[...] = jnp.where(upd, sz, az_ref[...])

    @pl.when(jnp.logical_and(q == 1, b == _NR - 1))
    def _():
        idx_ref[...] = jnp.broadcast_to(ai_ref[...], (8, _NR))
        cx_ref[...] = jnp.broadcast_to(ax_ref[...], (8, _NR))
        cy_ref[...] = jnp.broadcast_to(ay_ref[...], (8, _NR))
        cz_ref[...] = jnp.broadcast_to(az_ref[...], (8, _NR))


_extract_call = pl.pallas_call(
    _extract_body,
    grid_spec=pltpu.PrefetchScalarGridSpec(
        num_scalar_prefetch=1,
        grid=(2, _NR),
        in_specs=[
            pl.BlockSpec((3, _CL), lambda q, b, kbc: (0, kbc[q * _NR + b])),
            pl.BlockSpec((8, _NR), lambda q, b, kbc: (0, 0)),
        ],
        out_specs=[
            pl.BlockSpec((8, _NR), lambda q, b, kbc: (0, 0)),
            pl.BlockSpec((8, _NR), lambda q, b, kbc: (0, 0)),
            pl.BlockSpec((8, _NR), lambda q, b, kbc: (0, 0)),
            pl.BlockSpec((8, _NR), lambda q, b, kbc: (0, 0)),
        ],
        scratch_shapes=[
            pltpu.VMEM((1, _NR), jnp.float32),
            pltpu.VMEM((1, _NR), jnp.int32),
            pltpu.VMEM((1, _NR), jnp.float32),
            pltpu.VMEM((1, _NR), jnp.float32),
            pltpu.VMEM((1, _NR), jnp.float32),
        ],
    ),
    out_shape=[
        jax.ShapeDtypeStruct((8, _NR), jnp.int32),
        jax.ShapeDtypeStruct((8, _NR), jnp.float32),
        jax.ShapeDtypeStruct((8, _NR), jnp.float32),
        jax.ShapeDtypeStruct((8, _NR), jnp.float32),
    ],
)


def _sc_scatter_body(idx_hbm, col_ref, idx_v, buf_v, sem):
    wid = lax.axis_index("c") * 16 + lax.axis_index("s")

    @pl.when(wid == 0)
    def _():
        pltpu.sync_copy(idx_hbm, idx_v)
        vecs = [idx_v[pl.ds(0, 16)], idx_v[pl.ds(16, 16)]]
        iota16 = lax.iota(jnp.int32, 16)
        starts = []
        for j in range(_NR):
            idx_j = vecs[j // 16][j % 16]
            s_j = jnp.minimum((idx_j // 8) * 8, _L - 16)
            v = jnp.zeros((16,), jnp.float32)
            for k in range(_NR):
                off = vecs[k // 16][k % 16] - s_j
                v = jnp.where(iota16 == off, 1.0, v)
            buf_v[pl.ds(16 * j, 16)] = v
            starts.append(s_j)
        copies = []
        for j in range(_NR):
            copies.append(pltpu.async_copy(
                buf_v.at[pl.ds(16 * j, 16)],
                col_ref.at[pl.ds(starts[j], 16)],
                sem))
        for cc in copies:
            cc.wait()


@functools.lru_cache(maxsize=None)
def _make_sc_scatter():
    mesh = plsc.VectorSubcoreMesh(core_axis_name="c", subcore_axis_name="s")
    return pl.kernel(
        _sc_scatter_body,
        out_type=(),
        mesh=mesh,
        scratch_types=[
            pltpu.VMEM((_NR,), jnp.int32),
            pltpu.VMEM((16 * _NR,), jnp.float32),
            pltpu.SemaphoreType.DMA,
        ],
    )


def kernel(mesh_3D, receiver_pos):
    mt = mesh_3D.transpose((3, 0, 1, 2)).reshape(3, _L)
    r = receiver_pos.astype(jnp.float32)
    w8 = jnp.concatenate(
        [-2.0 * r, jnp.ones((_NR, 3), jnp.float32),
         jnp.zeros((_NR, 2), jnp.float32)], axis=1)
    rt = jnp.zeros((8, _NR), jnp.float32).at[0:3, :].set(r.T)
    kb1_8, kb2_8 = _argmin_call(mt, w8)
    kbc = jnp.concatenate([kb1_8[:, 0], kb2_8[:, 0]])
    idx8, cx8, cy8, cz8 = _extract_call(kbc, mt, rt)
    idx = idx8[0]
    closest = jnp.stack([cx8[0], cy8[0], cz8[0]], axis=1)
    col_ref = jax.new_ref(jnp.zeros((_L,), jnp.float32))
    _make_sc_scatter()(idx, col_ref)
    one_hot = col_ref[...]
    input_tensor = jnp.stack([mt[0], mt[1], mt[2], one_hot], axis=1)
    return input_tensor, closest


# speedup vs baseline: 15.8968x; 1.0752x over previous
import functools

import jax
import jax.numpy as jnp
from jax import lax
from jax.experimental import pallas as pl
from jax.experimental.pallas import tpu as pltpu
from jax.experimental.pallas import tpu_sc as plsc

_L = 1_000_000
_NR = 32
_BL = 32_768
_NBA = -(-_L // _BL)
_CL = 2_048
_CPB = _BL // _CL
_PREC = lax.Precision.HIGHEST

_BIG_I = 2**30
_INF = float("inf")
_FAR = 3.0e9


def _argmin_body(mt_ref, w_ref, kb1_ref, kb2_ref,
                 b1v_ref, b1c_ref, b2v_ref, b2c_ref):
    step = pl.program_id(0)

    @pl.when(step == 0)
    def _():
        b1v_ref[...] = jnp.full((_NR, 8), _INF, jnp.float32)
        b1c_ref[...] = jnp.zeros((_NR, 8), jnp.int32)
        b2v_ref[...] = jnp.full((_NR, 8), _INF, jnp.float32)
        b2c_ref[...] = jnp.zeros((_NR, 8), jnp.int32)

    lane = lax.broadcasted_iota(jnp.int32, (3, _BL), 1)
    ok = lane + step * _BL < _L
    blk = jnp.where(ok, mt_ref[...], _FAR)
    p6 = jnp.concatenate([blk, blk * blk], axis=0)
    s = jnp.dot(w_ref[:, 0:6], p6, precision=_PREC,
                preferred_element_type=jnp.float32)
    for j in range(_CPB):
        m = jnp.min(s[:, j * _CL:(j + 1) * _CL], axis=1,
                    keepdims=True)
        m = jnp.broadcast_to(m, (_NR, 8))
        cid = step * _CPB + j
        is1 = m < b1v_ref[...]
        is2 = m < b2v_ref[...]
        b2v_ref[...] = jnp.where(is1, b1v_ref[...],
                                 jnp.where(is2, m, b2v_ref[...]))
        b2c_ref[...] = jnp.where(is1, b1c_ref[...],
                                 jnp.where(is2, cid, b2c_ref[...]))
        b1v_ref[...] = jnp.where(is1, m, b1v_ref[...])
        b1c_ref[...] = jnp.where(is1, cid, b1c_ref[...])

    @pl.when(step == _NBA - 1)
    def _():
        kb1_ref[...] = b1c_ref[...]
        kb2_ref[...] = b2c_ref[...]


_argmin_call = pl.pallas_call(
    _argmin_body,
    grid=(_NBA,),
    in_specs=[
        pl.BlockSpec((3, _BL), lambda i: (0, i)),
        pl.BlockSpec((_NR, 8), lambda i: (0, 0)),
    ],
    out_specs=[
        pl.BlockSpec((_NR, 8), lambda i: (0, 0)),
        pl.BlockSpec((_NR, 8), lambda i: (0, 0)),
    ],
    out_shape=[
        jax.ShapeDtypeStruct((_NR, 8), jnp.int32),
        jax.ShapeDtypeStruct((_NR, 8), jnp.int32),
    ],
    scratch_shapes=[
        pltpu.VMEM((_NR, 8), jnp.float32),
        pltpu.VMEM((_NR, 8), jnp.int32),
        pltpu.VMEM((_NR, 8), jnp.float32),
        pltpu.VMEM((_NR, 8), jnp.int32),
    ],
)


def _extract_body(kbc_ref, mt1_ref, mt2_ref, rt_ref, idx_ref, cx_ref,
                  cy_ref, cz_ref, ai_ref, ax_ref, ay_ref, az_ref):
    b = pl.program_id(0)

    lane = lax.broadcasted_iota(jnp.int32, (3, _CL), 1)
    lane32 = lax.broadcasted_iota(jnp.int32, (8, _NR), 1)
    rcol = jnp.sum(jnp.where(lane32 == b, rt_ref[...], 0.0),
                   axis=1, keepdims=True)[0:3]
    zero = jnp.zeros((), jnp.float32)

    def scan(blk_ref, kblk):
        blk = blk_ref[...]
        dd = blk - rcol
        d2 = dd * dd
        d2p = (d2[0:1] + d2[1:2]) + d2[2:3]
        d2p = jnp.where(lane[0:1] + kblk * _CL < _L, d2p, _INF)
        m = jnp.min(d2p)
        flat = jnp.min(jnp.where(d2p == m, lane[0:1], _BIG_I))
        gidx = kblk * _CL + flat
        wsel = lane[0:1] == flat
        sx = jnp.sum(jnp.where(wsel, blk[0:1], zero))
        sy = jnp.sum(jnp.where(wsel, blk[1:2], zero))
        sz = jnp.sum(jnp.where(wsel, blk[2:3], zero))
        return m, gidx, sx, sy, sz

    m1, g1, x1, y1, z1 = scan(mt1_ref, kbc_ref[b])
    m2, g2, x2, y2, z2 = scan(mt2_ref, kbc_ref[_NR + b])
    take2 = jnp.logical_or(m2 < m1, jnp.logical_and(m2 == m1, g2 < g1))
    m = jnp.where(take2, m2, m1)
    gidx = jnp.where(take2, g2, g1)
    sx = jnp.where(take2, x2, x1)
    sy = jnp.where(take2, y2, y1)
    sz = jnp.where(take2, z2, z1)
    del m
    lane1 = lax.broadcasted_iota(jnp.int32, (1, _NR), 1)
    isb = lane1 == b
    ai_ref[...] = jnp.where(isb, gidx, ai_ref[...])
    ax_ref[...] = jnp.where(isb, sx, ax_ref[...])
    ay_ref[...] = jnp.where(isb, sy, ay_ref[...])
    az_ref[...] = jnp.where(isb, sz, az_ref[...])

    @pl.when(b == _NR - 1)
    def _():
        idx_ref[...] = jnp.broadcast_to(ai_ref[...], (8, _NR))
        cx_ref[...] = jnp.broadcast_to(ax_ref[...], (8, _NR))
        cy_ref[...] = jnp.broadcast_to(ay_ref[...], (8, _NR))
        cz_ref[...] = jnp.broadcast_to(az_ref[...], (8, _NR))


_extract_call = pl.pallas_call(
    _extract_body,
    grid_spec=pltpu.PrefetchScalarGridSpec(
        num_scalar_prefetch=1,
        grid=(_NR,),
        in_specs=[
            pl.BlockSpec((3, _CL), lambda b, kbc: (0, kbc[b])),
            pl.BlockSpec((3, _CL), lambda b, kbc: (0, kbc[_NR + b])),
            pl.BlockSpec((8, _NR), lambda b, kbc: (0, 0)),
        ],
        out_specs=[
            pl.BlockSpec((8, _NR), lambda b, kbc: (0, 0)),
            pl.BlockSpec((8, _NR), lambda b, kbc: (0, 0)),
            pl.BlockSpec((8, _NR), lambda b, kbc: (0, 0)),
            pl.BlockSpec((8, _NR), lambda b, kbc: (0, 0)),
        ],
        scratch_shapes=[
            pltpu.VMEM((1, _NR), jnp.int32),
            pltpu.VMEM((1, _NR), jnp.float32),
            pltpu.VMEM((1, _NR), jnp.float32),
            pltpu.VMEM((1, _NR), jnp.float32),
        ],
    ),
    out_shape=[
        jax.ShapeDtypeStruct((8, _NR), jnp.int32),
        jax.ShapeDtypeStruct((8, _NR), jnp.float32),
        jax.ShapeDtypeStruct((8, _NR), jnp.float32),
        jax.ShapeDtypeStruct((8, _NR), jnp.float32),
    ],
)


def _sc_scatter_body(idx_hbm, col_ref, idx_v, buf_v, sem):
    wid = lax.axis_index("c") * 16 + lax.axis_index("s")

    @pl.when(wid == 0)
    def _():
        pltpu.sync_copy(idx_hbm, idx_v)
        vecs = [idx_v[pl.ds(0, 16)], idx_v[pl.ds(16, 16)]]
        iota16 = lax.iota(jnp.int32, 16)
        starts = []
        for j in range(_NR):
            idx_j = vecs[j // 16][j % 16]
            s_j = jnp.minimum((idx_j // 8) * 8, _L - 16)
            v = jnp.zeros((16,), jnp.float32)
            for k in range(_NR):
                off = vecs[k // 16][k % 16] - s_j
                v = jnp.where(iota16 == off, 1.0, v)
            buf_v[pl.ds(16 * j, 16)] = v
            starts.append(s_j)
        copies = []
        for j in range(_NR):
            copies.append(pltpu.async_copy(
                buf_v.at[pl.ds(16 * j, 16)],
                col_ref.at[pl.ds(starts[j], 16)],
                sem))
        for cc in copies:
            cc.wait()


@functools.lru_cache(maxsize=None)
def _make_sc_scatter():
    mesh = plsc.VectorSubcoreMesh(core_axis_name="c", subcore_axis_name="s")
    return pl.kernel(
        _sc_scatter_body,
        out_type=(),
        mesh=mesh,
        scratch_types=[
            pltpu.VMEM((_NR,), jnp.int32),
            pltpu.VMEM((16 * _NR,), jnp.float32),
            pltpu.SemaphoreType.DMA,
        ],
    )


def kernel(mesh_3D, receiver_pos):
    mt = mesh_3D.transpose((3, 0, 1, 2)).reshape(3, _L)
    r = receiver_pos.astype(jnp.float32)
    w8 = jnp.concatenate(
        [-2.0 * r, jnp.ones((_NR, 3), jnp.float32),
         jnp.zeros((_NR, 2), jnp.float32)], axis=1)
    rt = jnp.zeros((8, _NR), jnp.float32).at[0:3, :].set(r.T)
    kb1_8, kb2_8 = _argmin_call(mt, w8)
    kbc = jnp.concatenate([kb1_8[:, 0], kb2_8[:, 0]])
    idx8, cx8, cy8, cz8 = _extract_call(kbc, mt, mt, rt)
    idx = idx8[0]
    closest = jnp.stack([cx8[0], cy8[0], cz8[0]], axis=1)
    col_ref = jax.new_ref(jnp.zeros((_L,), jnp.float32))
    _make_sc_scatter()(idx, col_ref)
    one_hot = col_ref[...]
    input_tensor = jnp.stack([mt[0], mt[1], mt[2], one_hot], axis=1)
    return input_tensor, closest


# speedup vs baseline: 16.1517x; 1.0160x over previous
import functools

import jax
import jax.numpy as jnp
from jax import lax
from jax.experimental import pallas as pl
from jax.experimental.pallas import tpu as pltpu
from jax.experimental.pallas import tpu_sc as plsc

_L = 1_000_000
_NR = 32
_BL = 65_536
_NBA = -(-_L // _BL)
_CL = 2_048
_CPB = _BL // _CL
_PREC = lax.Precision.HIGHEST

_BIG_I = 2**30
_INF = float("inf")
_FAR = 3.0e9


def _argmin_body(mt_ref, w_ref, kb1_ref, kb2_ref,
                 b1v_ref, b1c_ref, b2v_ref, b2c_ref):
    step = pl.program_id(0)

    @pl.when(step == 0)
    def _():
        b1v_ref[...] = jnp.full((_NR, 8), _INF, jnp.float32)
        b1c_ref[...] = jnp.zeros((_NR, 8), jnp.int32)
        b2v_ref[...] = jnp.full((_NR, 8), _INF, jnp.float32)
        b2c_ref[...] = jnp.zeros((_NR, 8), jnp.int32)

    lane = lax.broadcasted_iota(jnp.int32, (3, _BL), 1)
    ok = lane < _L - step * _BL
    blk = jnp.where(ok, mt_ref[...], _FAR)
    p6 = jnp.concatenate([blk, blk * blk], axis=0)
    s = jnp.dot(w_ref[:, 0:6], p6, precision=_PREC,
                preferred_element_type=jnp.float32)
    for j in range(_CPB):
        m = jnp.min(s[:, j * _CL:(j + 1) * _CL], axis=1,
                    keepdims=True)
        m = jnp.broadcast_to(m, (_NR, 8))
        cid = step * _CPB + j
        is1 = m < b1v_ref[...]
        is2 = m < b2v_ref[...]
        b2v_ref[...] = jnp.where(is1, b1v_ref[...],
                                 jnp.where(is2, m, b2v_ref[...]))
        b2c_ref[...] = jnp.where(is1, b1c_ref[...],
                                 jnp.where(is2, cid, b2c_ref[...]))
        b1v_ref[...] = jnp.where(is1, m, b1v_ref[...])
        b1c_ref[...] = jnp.where(is1, cid, b1c_ref[...])

    @pl.when(step == _NBA - 1)
    def _():
        kb1_ref[...] = b1c_ref[...]
        kb2_ref[...] = b2c_ref[...]


_argmin_call = pl.pallas_call(
    _argmin_body,
    grid=(_NBA,),
    in_specs=[
        pl.BlockSpec((3, _BL), lambda i: (0, i)),
        pl.BlockSpec((_NR, 8), lambda i: (0, 0)),
    ],
    out_specs=[
        pl.BlockSpec((_NR, 8), lambda i: (0, 0)),
        pl.BlockSpec((_NR, 8), lambda i: (0, 0)),
    ],
    out_shape=[
        jax.ShapeDtypeStruct((_NR, 8), jnp.int32),
        jax.ShapeDtypeStruct((_NR, 8), jnp.int32),
    ],
    scratch_shapes=[
        pltpu.VMEM((_NR, 8), jnp.float32),
        pltpu.VMEM((_NR, 8), jnp.int32),
        pltpu.VMEM((_NR, 8), jnp.float32),
        pltpu.VMEM((_NR, 8), jnp.int32),
    ],
)


def _extract_body(kbc_ref, c0_ref, c1_ref, c2_ref, c3_ref, c4_ref, c5_ref,
                  c6_ref, c7_ref, rt_ref, idx_ref, cx_ref, cy_ref, cz_ref,
                  ai_ref, ax_ref, ay_ref, az_ref):
    g = pl.program_id(0)
    chunk_refs = (c0_ref, c1_ref, c2_ref, c3_ref,
                  c4_ref, c5_ref, c6_ref, c7_ref)
    lane = lax.broadcasted_iota(jnp.int32, (3, _CL), 1)
    lane32 = lax.broadcasted_iota(jnp.int32, (8, _NR), 1)
    lane1 = lax.broadcasted_iota(jnp.int32, (1, _NR), 1)
    zero = jnp.zeros((), jnp.float32)

    def scan(blk_ref, kblk, rcol):
        blk = blk_ref[...]
        dd = blk - rcol
        d2 = dd * dd
        d2p = (d2[0:1] + d2[1:2]) + d2[2:3]
        d2p = jnp.where(lane[0:1] + kblk * _CL < _L, d2p, _INF)
        m = jnp.min(d2p)
        flat = jnp.min(jnp.where(d2p == m, lane[0:1], _BIG_I))
        gidx = kblk * _CL + flat
        wsel = lane[0:1] == flat
        sx = jnp.sum(jnp.where(wsel, blk[0:1], zero))
        sy = jnp.sum(jnp.where(wsel, blk[1:2], zero))
        sz = jnp.sum(jnp.where(wsel, blk[2:3], zero))
        return m, gidx, sx, sy, sz

    for i in range(4):
        b = g * 4 + i
        rcol = jnp.sum(jnp.where(lane32 == b, rt_ref[...], 0.0),
                       axis=1, keepdims=True)[0:3]
        m1, g1, x1, y1, z1 = scan(chunk_refs[2 * i], kbc_ref[b], rcol)
        m2, g2, x2, y2, z2 = scan(chunk_refs[2 * i + 1],
                                  kbc_ref[_NR + b], rcol)
        take2 = jnp.logical_or(m2 < m1,
                               jnp.logical_and(m2 == m1, g2 < g1))
        gidx = jnp.where(take2, g2, g1)
        sx = jnp.where(take2, x2, x1)
        sy = jnp.where(take2, y2, y1)
        sz = jnp.where(take2, z2, z1)
        isb = lane1 == b
        ai_ref[...] = jnp.where(isb, gidx, ai_ref[...])
        ax_ref[...] = jnp.where(isb, sx, ax_ref[...])
        ay_ref[...] = jnp.where(isb, sy, ay_ref[...])
        az_ref[...] = jnp.where(isb, sz, az_ref[...])

    @pl.when(g == _NR // 4 - 1)
    def _():
        idx_ref[...] = jnp.broadcast_to(ai_ref[...], (8, _NR))
        cx_ref[...] = jnp.broadcast_to(ax_ref[...], (8, _NR))
        cy_ref[...] = jnp.broadcast_to(ay_ref[...], (8, _NR))
        cz_ref[...] = jnp.broadcast_to(az_ref[...], (8, _NR))


def _chunk_spec(i, q):
    return pl.BlockSpec(
        (3, _CL), lambda g, kbc, i=i, q=q: (0, kbc[_NR * q + 4 * g + i]))


_extract_call = pl.pallas_call(
    _extract_body,
    grid_spec=pltpu.PrefetchScalarGridSpec(
        num_scalar_prefetch=1,
        grid=(_NR // 4,),
        in_specs=(
            [_chunk_spec(i, q) for i in range(4) for q in range(2)]
            + [pl.BlockSpec((8, _NR), lambda g, kbc: (0, 0))]),
        out_specs=[
            pl.BlockSpec((8, _NR), lambda g, kbc: (0, 0)),
            pl.BlockSpec((8, _NR), lambda g, kbc: (0, 0)),
            pl.BlockSpec((8, _NR), lambda g, kbc: (0, 0)),
            pl.BlockSpec((8, _NR), lambda g, kbc: (0, 0)),
        ],
        scratch_shapes=[
            pltpu.VMEM((1, _NR), jnp.int32),
            pltpu.VMEM((1, _NR), jnp.float32),
            pltpu.VMEM((1, _NR), jnp.float32),
            pltpu.VMEM((1, _NR), jnp.float32),
        ],
    ),
    out_shape=[
        jax.ShapeDtypeStruct((8, _NR), jnp.int32),
        jax.ShapeDtypeStruct((8, _NR), jnp.float32),
        jax.ShapeDtypeStruct((8, _NR), jnp.float32),
        jax.ShapeDtypeStruct((8, _NR), jnp.float32),
    ],
)


def _sc_scatter_body(idx_hbm, col_ref, idx_v, buf_v, sem):
    wid = lax.axis_index("c") * 16 + lax.axis_index("s")

    @pl.when(wid == 0)
    def _():
        pltpu.sync_copy(idx_hbm, idx_v)
        vecs = [idx_v[pl.ds(0, 16)], idx_v[pl.ds(16, 16)]]
        iota16 = lax.iota(jnp.int32, 16)
        starts = []
        for j in range(_NR):
            idx_j = vecs[j // 16][j % 16]
            s_j = jnp.minimum((idx_j // 8) * 8, _L - 16)
            v = jnp.zeros((16,), jnp.float32)
            for k in range(_NR):
                off = vecs[k // 16][k % 16] - s_j
                v = jnp.where(iota16 == off, 1.0, v)
            buf_v[pl.ds(16 * j, 16)] = v
            starts.append(s_j)
        copies = []
        for j in range(_NR):
            copies.append(pltpu.async_copy(
                buf_v.at[pl.ds(16 * j, 16)],
                col_ref.at[pl.ds(starts[j], 16)],
                sem))
        for cc in copies:
            cc.wait()


@functools.lru_cache(maxsize=None)
def _make_sc_scatter():
    mesh = plsc.VectorSubcoreMesh(core_axis_name="c", subcore_axis_name="s")
    return pl.kernel(
        _sc_scatter_body,
        out_type=(),
        mesh=mesh,
        scratch_types=[
            pltpu.VMEM((_NR,), jnp.int32),
            pltpu.VMEM((16 * _NR,), jnp.float32),
            pltpu.SemaphoreType.DMA,
        ],
    )


def kernel(mesh_3D, receiver_pos):
    mt = mesh_3D.transpose((3, 0, 1, 2)).reshape(3, _L)
    r = receiver_pos.astype(jnp.float32)
    w8 = jnp.concatenate(
        [-2.0 * r, jnp.ones((_NR, 3), jnp.float32),
         jnp.zeros((_NR, 2), jnp.float32)], axis=1)
    rt = jnp.zeros((8, _NR), jnp.float32).at[0:3, :].set(r.T)
    kb1_8, kb2_8 = _argmin_call(mt, w8)
    kbc = jnp.concatenate([kb1_8[:, 0], kb2_8[:, 0]])
    idx8, cx8, cy8, cz8 = _extract_call(kbc, *([mt] * 8), rt)
    idx = idx8[0]
    closest = jnp.stack([cx8[0], cy8[0], cz8[0]], axis=1)
    col_ref = jax.new_ref(jnp.zeros((_L,), jnp.float32))
    _make_sc_scatter()(idx, col_ref)
    one_hot = col_ref[...]
    input_tensor = jnp.stack([mt[0], mt[1], mt[2], one_hot], axis=1)
    return input_tensor, closest
